# Initial kernel scaffold; baseline (speedup 1.0000x reference)
import jax
import jax.numpy as jnp
from jax.experimental import pallas as pl


def kernel(x, levels):
    raise NotImplementedError("write your pallas kernel here")


# speedup vs baseline: 328.1770x; 328.1770x over previous
import functools

import jax
import jax.numpy as jnp
from jax import lax
from jax.experimental import pallas as pl
from jax.experimental.pallas import tpu as pltpu
from jax.experimental.pallas import tpu_sc as plsc

_L = 16
_NLEV = 16
_CHUNK = 16


def _sc_quantize(x2d, mids, gaps):
    nrows, d = x2d.shape
    nworkers = 32
    rows_per_w = nrows // nworkers
    nchunks = rows_per_w // _CHUNK
    nvec = d // _L

    mesh = plsc.VectorSubcoreMesh(core_axis_name="c", subcore_axis_name="s")

    @functools.partial(
        pl.kernel,
        mesh=mesh,
        out_type=jax.ShapeDtypeStruct((nrows, d), jnp.float32),
        compiler_params=pltpu.CompilerParams(needs_layout_passes=False),
        scratch_types=[
            pltpu.VMEM((_CHUNK, d), jnp.float32),
            pltpu.VMEM((_CHUNK, d), jnp.float32),
            pltpu.VMEM((_L,), jnp.float32),
            pltpu.VMEM((_L,), jnp.float32),
        ],
    )
    def k(x_hbm, mids_hbm, gaps_hbm, out_hbm, xbuf, obuf, mids_v, gaps_v):
        wid = lax.axis_index("s") * 2 + lax.axis_index("c")
        pltpu.sync_copy(mids_hbm, mids_v)
        pltpu.sync_copy(gaps_hbm, gaps_v)

        mvec = mids_v[...]
        gvec = gaps_v[...]
        mid_s = [jnp.full((_L,), mvec[i], jnp.float32) for i in range(_NLEV)]
        gap_s = [jnp.full((_L,), gvec[i], jnp.float32) for i in range(_NLEV)]

        def row_body(r, _):
            acc = jnp.zeros((_L,), jnp.float32)
            for j in range(nvec):
                v = xbuf[r, pl.ds(j * _L, _L)]
                acc = acc + v * v
            mean = jnp.sum(acc) * (1.0 / d)
            mv = jnp.full((_L,), mean, jnp.float32)
            bits = plsc.bitcast(mv, jnp.int32)
            y = plsc.bitcast((bits >> 1) + 0x1FBD1DF6, jnp.float32)
            y = 0.5 * (y + mv / y)
            y = 0.5 * (y + mv / y)
            y = 0.5 * (y + mv / y)
            stdv = y + 1e-8
            inv = 1.0 / stdv
            for j in range(nvec):
                v = xbuf[r, pl.ds(j * _L, _L)]
                t = v * inv
                q = jnp.zeros((_L,), jnp.float32)
                for i in range(_NLEV):
                    q = q + jnp.where(t > mid_s[i], gap_s[i], 0.0)
                obuf[r, pl.ds(j * _L, _L)] = q * stdv
            return _

        def chunk_body(c, _):
            base = wid * rows_per_w + c * _CHUNK
            pltpu.sync_copy(x_hbm.at[pl.ds(base, _CHUNK)], xbuf)
            lax.fori_loop(0, _CHUNK, row_body, 0)
            pltpu.sync_copy(obuf, out_hbm.at[pl.ds(base, _CHUNK)])
            return _

        lax.fori_loop(0, nchunks, chunk_body, 0)

    return k(x2d, mids, gaps)


def kernel(x, levels):
    lv = levels.astype(jnp.float32)
    mids = jnp.concatenate([jnp.array([-jnp.inf], jnp.float32),
                            0.5 * (lv[1:] + lv[:-1])])
    gaps = jnp.concatenate([lv[:1], lv[1:] - lv[:-1]])
    b, s, d = x.shape
    out = _sc_quantize(x.reshape(b * s, d), mids, gaps)
    return out.reshape(b, s, d)
